# baseline (speedup 1.0000x reference)
import functools

import jax
import jax.numpy as jnp
from jax import lax
from jax.experimental import pallas as pl
from jax.experimental.pallas import tpu as pltpu
from jax.experimental.pallas import tpu_sc as plsc

N = 1024
VOCAB = 100000
DIM = 64

INT_MAX = 2**31 - 1

TBV = 2560
TC_STEPS = 15


def _tc_body_impl(s_ref, a_ref, m_ref, blk_of):
    j = pl.program_id(0)
    blk = blk_of(j)
    col = lax.broadcasted_iota(jnp.int32, (N, TBV), 1) + blk * TBV
    v = jnp.where(col < VOCAB, s_ref[...], -jnp.inf)
    m = jnp.max(v, axis=1, keepdims=True)
    a = jnp.min(jnp.where(v == m, col, INT_MAX), axis=1, keepdims=True)

    @pl.when(j == 0)
    def _():
        m_ref[...] = m
        a_ref[...] = a

    @pl.when(j > 0)
    def _():
        better = m > m_ref[...]
        m_ref[...] = jnp.where(better, m, m_ref[...])
        a_ref[...] = jnp.where(better, a, a_ref[...])


def _tc_body(s_ref, a_ref, m_ref):
    _tc_body_impl(s_ref, a_ref, m_ref,
                  lambda j: jnp.where(j < TC_STEPS - 1, j, 39))


def _tc_body_all(s_ref, a_ref, m_ref):
    _tc_body_impl(s_ref, a_ref, m_ref, lambda j: j)


_tc_scan = pl.pallas_call(
    _tc_body,
    grid=(TC_STEPS,),
    in_specs=[pl.BlockSpec((N, TBV),
                           lambda j: (0, jnp.where(j < TC_STEPS - 1, j, 39)))],
    out_specs=(pl.BlockSpec((N, 1), lambda j: (0, 0)),
               pl.BlockSpec((N, 1), lambda j: (0, 0))),
    out_shape=(jax.ShapeDtypeStruct((N, 1), jnp.int32),
               jax.ShapeDtypeStruct((N, 1), jnp.float32)),
    compiler_params=pltpu.CompilerParams(
        dimension_semantics=("arbitrary",),
    ),
)

NC, NS = 2, 16
NW = NC * NS
RW = N // NW
VT_SC = 35456
W = 64512
CW = 3584
CHUNKS = W // CW
PAIRS = CHUNKS // 2
GROUPS = CW // 64


def _lex(m0, c0, m1, c1):
    take1 = (m1 > m0) | ((m1 == m0) & (c1 < c0))
    return jnp.where(take1, m1, m0), jnp.where(take1, c1, c0)


@functools.cache
def _make_scan_sc():
    mesh = plsc.VectorSubcoreMesh(core_axis_name="c", subcore_axis_name="s")

    @functools.partial(
        pl.kernel,
        mesh=mesh,
        out_type=(jax.ShapeDtypeStruct((N * 16,), jnp.float32),
                  jax.ShapeDtypeStruct((N * 16,), jnp.int32)),
        scratch_types=[
            pltpu.VMEM((8, CW), jnp.float32),
            pltpu.VMEM((8, CW), jnp.float32),
            pltpu.VMEM((512,), jnp.float32),
            pltpu.VMEM((512,), jnp.int32),
            pltpu.VMEM((512,), jnp.float32),
            pltpu.VMEM((512,), jnp.int32),
            pltpu.SemaphoreType.DMA,
            pltpu.SemaphoreType.DMA,
        ],
        compiler_params=pltpu.CompilerParams(use_tc_tiling_on_sc=True),
    )
    def _scan(x_hbm, mout_hbm, cout_hbm,
              buf_a, buf_b, m_st, a_st, m_stage, c_stage, sem_a, sem_b):
        wid = lax.axis_index("s") * NC + lax.axis_index("c")
        lane = lax.iota(jnp.int32, 16)
        neg_inf = jnp.full((16,), -jnp.inf, jnp.float32)
        zero16 = jnp.zeros((16,), jnp.int32)

        def src(oct_, c):
            row0 = wid * RW + oct_ * 8
            return x_hbm.at[pl.ds(row0, 8), pl.ds(VT_SC + c * CW, CW)]

        def process(c, buf):
            for r in range(8):
                carry0 = (tuple(m_st[pl.ds((r * 4 + k) * 16, 16)]
                                for k in range(4))
                          + tuple(a_st[pl.ds((r * 4 + k) * 16, 16)]
                                  for k in range(4)))

                def body(g, cr, r=r, buf=buf, c=c):
                    ms = list(cr[:4])
                    as_ = list(cr[4:])
                    gg = jnp.broadcast_to(c * GROUPS + g, (16,))
                    for k in range(4):
                        v = buf[r, pl.ds(g * 64 + k * 16, 16)]
                        upd = v > ms[k]
                        ms[k] = jnp.where(upd, v, ms[k])
                        as_[k] = jnp.where(upd, gg, as_[k])
                    return tuple(ms) + tuple(as_)

                fin = plsc.parallel_loop(0, GROUPS, carry=carry0)(body)
                for k in range(4):
                    m_st[pl.ds((r * 4 + k) * 16, 16)] = fin[k]
                    a_st[pl.ds((r * 4 + k) * 16, 16)] = fin[4 + k]

        def octet_body(oct_, _):
            def init_body(i, c):
                m_st[pl.ds(i * 16, 16)] = neg_inf
                a_st[pl.ds(i * 16, 16)] = zero16
                return c
            lax.fori_loop(0, 32, init_body, 0)

            pltpu.async_copy(src(oct_, 0), buf_a, sem_a)

            def pair_body(p, c):
                c0 = 2 * p
                cp_b = pltpu.async_copy(src(oct_, c0 + 1), buf_b, sem_b)
                pltpu.make_async_copy(src(oct_, c0), buf_a, sem_a).wait()
                process(c0, buf_a)

                @pl.when(p < PAIRS - 1)
                def _():
                    pltpu.async_copy(src(oct_, c0 + 2), buf_a, sem_a)

                cp_b.wait()
                process(c0 + 1, buf_b)
                return c
            lax.fori_loop(0, PAIRS, pair_body, 0)

            for r in range(8):
                mk = [m_st[pl.ds((r * 4 + k) * 16, 16)] for k in range(4)]
                ck = [VT_SC + a_st[pl.ds((r * 4 + k) * 16, 16)] * 64
                      + k * 16 + lane for k in range(4)]
                m01, c01 = _lex(mk[0], ck[0], mk[1], ck[1])
                m23, c23 = _lex(mk[2], ck[2], mk[3], ck[3])
                mf, cf = _lex(m01, c01, m23, c23)
                off = (oct_ * 8 + r) * 16
                m_stage[pl.ds(off, 16)] = mf
                c_stage[pl.ds(off, 16)] = cf
            return _

        lax.fori_loop(0, 4, octet_body, 0)
        pltpu.sync_copy(m_stage, mout_hbm.at[pl.ds(wid * 512, 512)])
        pltpu.sync_copy(c_stage, cout_hbm.at[pl.ds(wid * 512, 512)])

    return _scan


def _merge_body(mt_ref, at_ref, ml_ref, cl_ref, best_ref):
    m_t = mt_ref[...]
    a_t = at_ref[...]
    ml = ml_ref[...]
    cl = cl_ref[...]
    g = jnp.max(ml, axis=1, keepdims=True)
    c_s = jnp.min(jnp.where(ml == g, cl, INT_MAX), axis=1, keepdims=True)
    take = (g > m_t) | ((g == m_t) & (c_s < a_t))
    best_ref[...] = jnp.where(take, c_s, a_t)


_merge_call = pl.pallas_call(
    _merge_body,
    out_shape=jax.ShapeDtypeStruct((N, 1), jnp.int32),
)


BPW = N // NW


@functools.cache
def _make_gather_sc():
    mesh = plsc.VectorSubcoreMesh(core_axis_name="c", subcore_axis_name="s")

    @functools.partial(
        pl.kernel,
        mesh=mesh,
        out_type=jax.ShapeDtypeStruct((N, DIM), jnp.float32),
        scratch_types=[
            pltpu.VMEM((BPW,), jnp.int32),
            pltpu.VMEM((BPW, DIM), jnp.float32),
            pltpu.SemaphoreType.DMA,
        ],
        compiler_params=pltpu.CompilerParams(use_tc_tiling_on_sc=False),
    )
    def _gather_sc(table_hbm, idx_hbm, out_hbm, idx_v, rows_v, sem):
        wid = lax.axis_index("s") * NC + lax.axis_index("c")
        base = wid * BPW
        pltpu.sync_copy(idx_hbm.at[pl.ds(base, BPW)], idx_v)
        pltpu.async_copy(table_hbm.at[idx_v], rows_v, sem).wait()
        pltpu.sync_copy(rows_v, out_hbm.at[pl.ds(base, BPW)])

    return _gather_sc


_tc_scan_full = pl.pallas_call(
    _tc_body_all,
    grid=(40,),
    in_specs=[pl.BlockSpec((N, TBV), lambda j: (0, j))],
    out_specs=(pl.BlockSpec((N, 1), lambda j: (0, 0)),
               pl.BlockSpec((N, 1), lambda j: (0, 0))),
    out_shape=(jax.ShapeDtypeStruct((N, 1), jnp.int32),
               jax.ShapeDtypeStruct((N, 1), jnp.float32)),
    compiler_params=pltpu.CompilerParams(
        dimension_semantics=("arbitrary",),
    ),
)


def kernel(scores, emb_weight):
    a_t, m_t = _tc_scan_full(scores)
    best = a_t.reshape(N)
    emb = _make_gather_sc()(emb_weight, best)
    return emb, best


# speedup vs baseline: 2.6415x; 2.6415x over previous
import functools

import jax
import jax.numpy as jnp
from jax import lax
from jax.experimental import pallas as pl
from jax.experimental.pallas import tpu as pltpu
from jax.experimental.pallas import tpu_sc as plsc

N = 1024
VOCAB = 100000
DIM = 64

INT_MAX = 2**31 - 1

TBV = 2560
TC_STEPS = 15


def _tc_body_impl(s_ref, a_ref, m_ref, blk_of):
    j = pl.program_id(0)
    blk = blk_of(j)
    col = lax.broadcasted_iota(jnp.int32, (N, TBV), 1) + blk * TBV
    v = jnp.where(col < VOCAB, s_ref[...], -jnp.inf)
    m = jnp.max(v, axis=1, keepdims=True)
    a = jnp.min(jnp.where(v == m, col, INT_MAX), axis=1, keepdims=True)

    @pl.when(j == 0)
    def _():
        m_ref[...] = m
        a_ref[...] = a

    @pl.when(j > 0)
    def _():
        better = m > m_ref[...]
        m_ref[...] = jnp.where(better, m, m_ref[...])
        a_ref[...] = jnp.where(better, a, a_ref[...])


def _tc_body(s_ref, a_ref, m_ref):
    _tc_body_impl(s_ref, a_ref, m_ref,
                  lambda j: jnp.where(j < TC_STEPS - 1, j, 39))


def _tc_body_all(s_ref, a_ref, m_ref):
    _tc_body_impl(s_ref, a_ref, m_ref, lambda j: j)


_tc_scan = pl.pallas_call(
    _tc_body,
    grid=(TC_STEPS,),
    in_specs=[pl.BlockSpec((N, TBV),
                           lambda j: (0, jnp.where(j < TC_STEPS - 1, j, 39)))],
    out_specs=(pl.BlockSpec((N, 1), lambda j: (0, 0)),
               pl.BlockSpec((N, 1), lambda j: (0, 0))),
    out_shape=(jax.ShapeDtypeStruct((N, 1), jnp.int32),
               jax.ShapeDtypeStruct((N, 1), jnp.float32)),
    compiler_params=pltpu.CompilerParams(
        dimension_semantics=("arbitrary",),
    ),
)

NC, NS = 2, 16
NW = NC * NS
RW = N // NW
VT_SC = 35456
W = 64512
CW = 3584
CHUNKS = W // CW
PAIRS = CHUNKS // 2
GROUPS = CW // 64


def _lex(m0, c0, m1, c1):
    take1 = (m1 > m0) | ((m1 == m0) & (c1 < c0))
    return jnp.where(take1, m1, m0), jnp.where(take1, c1, c0)


@functools.cache
def _make_scan_sc():
    mesh = plsc.VectorSubcoreMesh(core_axis_name="c", subcore_axis_name="s")

    @functools.partial(
        pl.kernel,
        mesh=mesh,
        out_type=(jax.ShapeDtypeStruct((N * 16,), jnp.float32),
                  jax.ShapeDtypeStruct((N * 16,), jnp.int32)),
        scratch_types=[
            pltpu.VMEM((8, CW), jnp.float32),
            pltpu.VMEM((8, CW), jnp.float32),
            pltpu.VMEM((512,), jnp.float32),
            pltpu.VMEM((512,), jnp.int32),
            pltpu.VMEM((512,), jnp.float32),
            pltpu.VMEM((512,), jnp.int32),
            pltpu.SemaphoreType.DMA,
            pltpu.SemaphoreType.DMA,
        ],
        compiler_params=pltpu.CompilerParams(use_tc_tiling_on_sc=True),
    )
    def _scan(x_hbm, mout_hbm, cout_hbm,
              buf_a, buf_b, m_st, a_st, m_stage, c_stage, sem_a, sem_b):
        wid = lax.axis_index("s") * NC + lax.axis_index("c")
        lane = lax.iota(jnp.int32, 16)
        neg_inf = jnp.full((16,), -jnp.inf, jnp.float32)
        zero16 = jnp.zeros((16,), jnp.int32)

        def src(oct_, c):
            row0 = wid * RW + oct_ * 8
            return x_hbm.at[pl.ds(row0, 8), pl.ds(VT_SC + c * CW, CW)]

        def process(c, buf):
            for r in range(8):
                carry0 = (tuple(m_st[pl.ds((r * 4 + k) * 16, 16)]
                                for k in range(4))
                          + tuple(a_st[pl.ds((r * 4 + k) * 16, 16)]
                                  for k in range(4)))

                def body(g, cr, r=r, buf=buf, c=c):
                    ms = list(cr[:4])
                    as_ = list(cr[4:])
                    gg = jnp.broadcast_to(c * GROUPS + g, (16,))
                    for k in range(4):
                        v = buf[r, pl.ds(g * 64 + k * 16, 16)]
                        upd = v > ms[k]
                        ms[k] = jnp.where(upd, v, ms[k])
                        as_[k] = jnp.where(upd, gg, as_[k])
                    return tuple(ms) + tuple(as_)

                fin = plsc.parallel_loop(0, GROUPS, carry=carry0)(body)
                for k in range(4):
                    m_st[pl.ds((r * 4 + k) * 16, 16)] = fin[k]
                    a_st[pl.ds((r * 4 + k) * 16, 16)] = fin[4 + k]

        def octet_body(oct_, _):
            def init_body(i, c):
                m_st[pl.ds(i * 16, 16)] = neg_inf
                a_st[pl.ds(i * 16, 16)] = zero16
                return c
            lax.fori_loop(0, 32, init_body, 0)

            pltpu.async_copy(src(oct_, 0), buf_a, sem_a)

            def pair_body(p, c):
                c0 = 2 * p
                cp_b = pltpu.async_copy(src(oct_, c0 + 1), buf_b, sem_b)
                pltpu.make_async_copy(src(oct_, c0), buf_a, sem_a).wait()
                process(c0, buf_a)

                @pl.when(p < PAIRS - 1)
                def _():
                    pltpu.async_copy(src(oct_, c0 + 2), buf_a, sem_a)

                cp_b.wait()
                process(c0 + 1, buf_b)
                return c
            lax.fori_loop(0, PAIRS, pair_body, 0)

            for r in range(8):
                mk = [m_st[pl.ds((r * 4 + k) * 16, 16)] for k in range(4)]
                ck = [VT_SC + a_st[pl.ds((r * 4 + k) * 16, 16)] * 64
                      + k * 16 + lane for k in range(4)]
                m01, c01 = _lex(mk[0], ck[0], mk[1], ck[1])
                m23, c23 = _lex(mk[2], ck[2], mk[3], ck[3])
                mf, cf = _lex(m01, c01, m23, c23)
                off = (oct_ * 8 + r) * 16
                m_stage[pl.ds(off, 16)] = mf
                c_stage[pl.ds(off, 16)] = cf
            return _

        lax.fori_loop(0, 4, octet_body, 0)
        pltpu.sync_copy(m_stage, mout_hbm.at[pl.ds(wid * 512, 512)])
        pltpu.sync_copy(c_stage, cout_hbm.at[pl.ds(wid * 512, 512)])

    return _scan


def _merge_body(mt_ref, at_ref, ml_ref, cl_ref, best_ref):
    m_t = mt_ref[...]
    a_t = at_ref[...]
    ml = ml_ref[...]
    cl = cl_ref[...]
    g = jnp.max(ml, axis=1, keepdims=True)
    c_s = jnp.min(jnp.where(ml == g, cl, INT_MAX), axis=1, keepdims=True)
    take = (g > m_t) | ((g == m_t) & (c_s < a_t))
    best_ref[...] = jnp.where(take, c_s, a_t)


_merge_call = pl.pallas_call(
    _merge_body,
    out_shape=jax.ShapeDtypeStruct((N, 1), jnp.int32),
)


BPW = N // NW


@functools.cache
def _make_gather_sc():
    mesh = plsc.VectorSubcoreMesh(core_axis_name="c", subcore_axis_name="s")

    @functools.partial(
        pl.kernel,
        mesh=mesh,
        out_type=jax.ShapeDtypeStruct((N, DIM), jnp.float32),
        scratch_types=[
            pltpu.VMEM((BPW,), jnp.int32),
            pltpu.VMEM((BPW, DIM), jnp.float32),
            pltpu.SemaphoreType.DMA,
        ],
        compiler_params=pltpu.CompilerParams(use_tc_tiling_on_sc=False),
    )
    def _gather_sc(table_hbm, idx_hbm, out_hbm, idx_v, rows_v, sem):
        wid = lax.axis_index("s") * NC + lax.axis_index("c")
        base = wid * BPW
        pltpu.sync_copy(idx_hbm.at[pl.ds(base, BPW)], idx_v)
        pltpu.async_copy(table_hbm.at[idx_v], rows_v, sem).wait()
        pltpu.sync_copy(rows_v, out_hbm.at[pl.ds(base, BPW)])

    return _gather_sc


TBVT = 2048
NBT = (VOCAB + TBVT - 1) // TBVT


def _tct_body(s_ref, a_ref, m_ref):
    j = pl.program_id(0)
    vidx = lax.broadcasted_iota(jnp.int32, (TBVT, N), 0) + j * TBVT
    v = jnp.where(vidx < VOCAB, s_ref[...], -jnp.inf)
    m = jnp.max(v, axis=0, keepdims=True)
    a = jnp.min(jnp.where(v == m, vidx, INT_MAX), axis=0, keepdims=True)

    @pl.when(j == 0)
    def _():
        m_ref[...] = m
        a_ref[...] = a

    @pl.when(j > 0)
    def _():
        better = m > m_ref[...]
        m_ref[...] = jnp.where(better, m, m_ref[...])
        a_ref[...] = jnp.where(better, a, a_ref[...])


_tct_scan = pl.pallas_call(
    _tct_body,
    grid=(NBT,),
    in_specs=[pl.BlockSpec((TBVT, N), lambda j: (j, 0))],
    out_specs=(pl.BlockSpec((1, N), lambda j: (0, 0)),
               pl.BlockSpec((1, N), lambda j: (0, 0))),
    out_shape=(jax.ShapeDtypeStruct((1, N), jnp.int32),
               jax.ShapeDtypeStruct((1, N), jnp.float32)),
    compiler_params=pltpu.CompilerParams(
        dimension_semantics=("arbitrary",),
    ),
)


def kernel(scores, emb_weight):
    a_t, m_t = _tct_scan(jnp.swapaxes(scores, 0, 1))
    best = a_t.reshape(N)
    emb = _make_gather_sc()(emb_weight, best)
    return emb, best


# speedup vs baseline: 2.6666x; 1.0095x over previous
import functools

import jax
import jax.numpy as jnp
from jax import lax
from jax.experimental import pallas as pl
from jax.experimental.pallas import tpu as pltpu
from jax.experimental.pallas import tpu_sc as plsc

N = 1024
VOCAB = 100000
DIM = 64

INT_MAX = 2**31 - 1

TBVT = 2048
TC_FULL = 30
TAIL_BLK = 48
TC_STEPS = TC_FULL + 1


def _tct_body(s_ref, a_ref, m_ref):
    j = pl.program_id(0)
    blk = jnp.where(j < TC_FULL, j, TAIL_BLK)
    vidx = lax.broadcasted_iota(jnp.int32, (TBVT, N), 0) + blk * TBVT
    v = jnp.where(vidx < VOCAB, s_ref[...], -jnp.inf)
    m = jnp.max(v, axis=0, keepdims=True)
    a = jnp.min(jnp.where(v == m, vidx, INT_MAX), axis=0, keepdims=True)

    @pl.when(j == 0)
    def _():
        m_ref[...] = m
        a_ref[...] = a

    @pl.when(j > 0)
    def _():
        better = m > m_ref[...]
        m_ref[...] = jnp.where(better, m, m_ref[...])
        a_ref[...] = jnp.where(better, a, a_ref[...])


_tct_scan = pl.pallas_call(
    _tct_body,
    grid=(TC_STEPS,),
    in_specs=[pl.BlockSpec((TBVT, N),
                           lambda j: (jnp.where(j < TC_FULL, j, TAIL_BLK), 0))],
    out_specs=(pl.BlockSpec((1, N), lambda j: (0, 0)),
               pl.BlockSpec((1, N), lambda j: (0, 0))),
    out_shape=(jax.ShapeDtypeStruct((1, N), jnp.int32),
               jax.ShapeDtypeStruct((1, N), jnp.float32)),
    compiler_params=pltpu.CompilerParams(
        dimension_semantics=("arbitrary",),
    ),
)

NC, NS = 2, 16
NW = NC * NS
SC_LO = TC_FULL * TBVT
WS = 1152
CR = 32
NCH = WS // CR
NPAIR = NCH // 2


@functools.cache
def _make_scan_sc():
    mesh = plsc.VectorSubcoreMesh(core_axis_name="c", subcore_axis_name="s")

    @functools.partial(
        pl.kernel,
        mesh=mesh,
        out_type=(jax.ShapeDtypeStruct((NW * N,), jnp.float32),
                  jax.ShapeDtypeStruct((NW * N,), jnp.int32)),
        scratch_types=[
            pltpu.VMEM((CR, N), jnp.float32),
            pltpu.VMEM((CR, N), jnp.float32),
            pltpu.VMEM((N,), jnp.float32),
            pltpu.VMEM((N,), jnp.int32),
            pltpu.SemaphoreType.DMA,
            pltpu.SemaphoreType.DMA,
        ],
        compiler_params=pltpu.CompilerParams(use_tc_tiling_on_sc=True),
    )
    def _scan(st_hbm, mout_hbm, aout_hbm,
              buf_a, buf_b, m_st, a_st, sem_a, sem_b):
        wid = lax.axis_index("s") * NC + lax.axis_index("c")
        stripe0 = SC_LO + wid * WS
        neg_inf = jnp.full((16,), -jnp.inf, jnp.float32)
        zero16 = jnp.zeros((16,), jnp.int32)

        def init_body(i, c):
            m_st[pl.ds(i * 16, 16)] = neg_inf
            a_st[pl.ds(i * 16, 16)] = zero16
            return c
        lax.fori_loop(0, N // 16, init_body, 0)

        def src(c):
            return st_hbm.at[pl.ds(stripe0 + c * CR, CR)]

        def process(c, buf):
            for cg4 in range(N // 64):
                carry0 = (tuple(m_st[pl.ds((cg4 * 4 + k) * 16, 16)]
                                for k in range(4))
                          + tuple(a_st[pl.ds((cg4 * 4 + k) * 16, 16)]
                                  for k in range(4)))

                def body(row, cr, cg4=cg4, buf=buf, c=c):
                    ms = list(cr[:4])
                    as_ = list(cr[4:])
                    vid = jnp.broadcast_to(stripe0 + c * CR + row, (16,))
                    for k in range(4):
                        v = buf[row, pl.ds((cg4 * 4 + k) * 16, 16)]
                        upd = v > ms[k]
                        ms[k] = jnp.where(upd, v, ms[k])
                        as_[k] = jnp.where(upd, vid, as_[k])
                    return tuple(ms) + tuple(as_)

                fin = plsc.parallel_loop(0, CR, carry=carry0)(body)
                for k in range(4):
                    m_st[pl.ds((cg4 * 4 + k) * 16, 16)] = fin[k]
                    a_st[pl.ds((cg4 * 4 + k) * 16, 16)] = fin[4 + k]

        pltpu.async_copy(src(0), buf_a, sem_a)

        def pair_body(p, c):
            c0 = 2 * p
            cp_b = pltpu.async_copy(src(c0 + 1), buf_b, sem_b)
            pltpu.make_async_copy(src(c0), buf_a, sem_a).wait()
            process(c0, buf_a)

            @pl.when(p < NPAIR - 1)
            def _():
                pltpu.async_copy(src(c0 + 2), buf_a, sem_a)

            cp_b.wait()
            process(c0 + 1, buf_b)
            return c
        lax.fori_loop(0, NPAIR, pair_body, 0)

        pltpu.sync_copy(m_st, mout_hbm.at[pl.ds(wid * N, N)])
        pltpu.sync_copy(a_st, aout_hbm.at[pl.ds(wid * N, N)])

    return _scan


def _merge_body(at_ref, mt_ref, mp_ref, ap_ref, best_ref):
    a_t = at_ref[...]
    m_t = mt_ref[...]
    mp = mp_ref[...]
    ap = ap_ref[...]
    m_s = jnp.max(mp, axis=0, keepdims=True)
    a_s = jnp.min(jnp.where(mp == m_s, ap, INT_MAX), axis=0, keepdims=True)
    take = (m_s > m_t) | ((m_s == m_t) & (a_s < a_t))
    best_ref[...] = jnp.where(take, a_s, a_t)


_merge_call = pl.pallas_call(
    _merge_body,
    out_shape=jax.ShapeDtypeStruct((1, N), jnp.int32),
)


BPW = N // NW


@functools.cache
def _make_gather_sc():
    mesh = plsc.VectorSubcoreMesh(core_axis_name="c", subcore_axis_name="s")

    @functools.partial(
        pl.kernel,
        mesh=mesh,
        out_type=jax.ShapeDtypeStruct((N, DIM), jnp.float32),
        scratch_types=[
            pltpu.VMEM((BPW,), jnp.int32),
            pltpu.VMEM((BPW, DIM), jnp.float32),
            pltpu.SemaphoreType.DMA,
        ],
        compiler_params=pltpu.CompilerParams(use_tc_tiling_on_sc=False),
    )
    def _gather_sc(table_hbm, idx_hbm, out_hbm, idx_v, rows_v, sem):
        wid = lax.axis_index("s") * NC + lax.axis_index("c")
        base = wid * BPW
        pltpu.sync_copy(idx_hbm.at[pl.ds(base, BPW)], idx_v)
        pltpu.async_copy(table_hbm.at[idx_v], rows_v, sem).wait()
        pltpu.sync_copy(rows_v, out_hbm.at[pl.ds(base, BPW)])

    return _gather_sc


def kernel(scores, emb_weight):
    st = jnp.swapaxes(scores, 0, 1)
    a_t, m_t = _tct_scan(st)
    m_flat, a_flat = _make_scan_sc()(st)
    mp = m_flat.reshape(NW, N)
    ap = a_flat.reshape(NW, N)
    best = _merge_call(a_t, m_t, mp, ap).reshape(N)
    emb = _make_gather_sc()(emb_weight, best)
    return emb, best


# speedup vs baseline: 2.8435x; 1.0663x over previous
import functools

import jax
import jax.numpy as jnp
from jax import lax
from jax.experimental import pallas as pl
from jax.experimental.pallas import tpu as pltpu
from jax.experimental.pallas import tpu_sc as plsc

N = 1024
VOCAB = 100000
DIM = 64

INT_MAX = 2**31 - 1

TBVT = 2048
TC_FULL = 25
TAIL_BLK = 48
TC_STEPS = TC_FULL + 1


def _tct_body(s_ref, a_ref, m_ref):
    j = pl.program_id(0)
    blk = jnp.where(j < TC_FULL, j, TAIL_BLK)
    vidx = lax.broadcasted_iota(jnp.int32, (TBVT, N), 0) + blk * TBVT
    v = jnp.where(vidx < VOCAB, s_ref[...], -jnp.inf)
    m = jnp.max(v, axis=0, keepdims=True)
    a = jnp.min(jnp.where(v == m, vidx, INT_MAX), axis=0, keepdims=True)

    @pl.when(j == 0)
    def _():
        m_ref[...] = m
        a_ref[...] = a

    @pl.when(j > 0)
    def _():
        better = m > m_ref[...]
        m_ref[...] = jnp.where(better, m, m_ref[...])
        a_ref[...] = jnp.where(better, a, a_ref[...])


_tct_scan = pl.pallas_call(
    _tct_body,
    grid=(TC_STEPS,),
    in_specs=[pl.BlockSpec((TBVT, N),
                           lambda j: (jnp.where(j < TC_FULL, j, TAIL_BLK), 0))],
    out_specs=(pl.BlockSpec((1, N), lambda j: (0, 0)),
               pl.BlockSpec((1, N), lambda j: (0, 0))),
    out_shape=(jax.ShapeDtypeStruct((1, N), jnp.int32),
               jax.ShapeDtypeStruct((1, N), jnp.float32)),
    compiler_params=pltpu.CompilerParams(
        dimension_semantics=("arbitrary",),
    ),
)

NC, NS = 2, 16
NW = NC * NS
SC_LO = TC_FULL * TBVT
WS = 1472
CR = 32
NCH = WS // CR
NPAIR = NCH // 2


@functools.cache
def _make_scan_sc():
    mesh = plsc.VectorSubcoreMesh(core_axis_name="c", subcore_axis_name="s")

    @functools.partial(
        pl.kernel,
        mesh=mesh,
        out_type=(jax.ShapeDtypeStruct((NW * N,), jnp.float32),
                  jax.ShapeDtypeStruct((NW * N,), jnp.int32)),
        scratch_types=[
            pltpu.VMEM((CR, N), jnp.float32),
            pltpu.VMEM((CR, N), jnp.float32),
            pltpu.VMEM((N,), jnp.float32),
            pltpu.VMEM((N,), jnp.int32),
            pltpu.SemaphoreType.DMA,
            pltpu.SemaphoreType.DMA,
        ],
        compiler_params=pltpu.CompilerParams(use_tc_tiling_on_sc=True),
    )
    def _scan(st_hbm, mout_hbm, aout_hbm,
              buf_a, buf_b, m_st, a_st, sem_a, sem_b):
        wid = lax.axis_index("s") * NC + lax.axis_index("c")
        stripe0 = SC_LO + wid * WS
        neg_inf = jnp.full((16,), -jnp.inf, jnp.float32)
        zero16 = jnp.zeros((16,), jnp.int32)

        def init_body(i, c):
            m_st[pl.ds(i * 16, 16)] = neg_inf
            a_st[pl.ds(i * 16, 16)] = zero16
            return c
        lax.fori_loop(0, N // 16, init_body, 0)

        def src(c):
            return st_hbm.at[pl.ds(stripe0 + c * CR, CR)]

        def process(c, buf):
            for cg4 in range(N // 64):
                carry0 = (tuple(m_st[pl.ds((cg4 * 4 + k) * 16, 16)]
                                for k in range(4))
                          + tuple(a_st[pl.ds((cg4 * 4 + k) * 16, 16)]
                                  for k in range(4)))

                def body(row, cr, cg4=cg4, buf=buf, c=c):
                    ms = list(cr[:4])
                    as_ = list(cr[4:])
                    vid = jnp.broadcast_to(stripe0 + c * CR + row, (16,))
                    for k in range(4):
                        v = buf[row, pl.ds((cg4 * 4 + k) * 16, 16)]
                        upd = v > ms[k]
                        ms[k] = jnp.where(upd, v, ms[k])
                        as_[k] = jnp.where(upd, vid, as_[k])
                    return tuple(ms) + tuple(as_)

                fin = plsc.parallel_loop(0, CR, carry=carry0)(body)
                for k in range(4):
                    m_st[pl.ds((cg4 * 4 + k) * 16, 16)] = fin[k]
                    a_st[pl.ds((cg4 * 4 + k) * 16, 16)] = fin[4 + k]

        pltpu.async_copy(src(0), buf_a, sem_a)

        def pair_body(p, c):
            c0 = 2 * p
            cp_b = pltpu.async_copy(src(c0 + 1), buf_b, sem_b)
            pltpu.make_async_copy(src(c0), buf_a, sem_a).wait()
            process(c0, buf_a)

            @pl.when(p < NPAIR - 1)
            def _():
                pltpu.async_copy(src(c0 + 2), buf_a, sem_a)

            cp_b.wait()
            process(c0 + 1, buf_b)
            return c
        lax.fori_loop(0, NPAIR, pair_body, 0)

        pltpu.sync_copy(m_st, mout_hbm.at[pl.ds(wid * N, N)])
        pltpu.sync_copy(a_st, aout_hbm.at[pl.ds(wid * N, N)])

    return _scan


def _merge_body(at_ref, mt_ref, mp_ref, ap_ref, best_ref):
    a_t = at_ref[...]
    m_t = mt_ref[...]
    mp = mp_ref[...]
    ap = ap_ref[...]
    m_s = jnp.max(mp, axis=0, keepdims=True)
    a_s = jnp.min(jnp.where(mp == m_s, ap, INT_MAX), axis=0, keepdims=True)
    take = (m_s > m_t) | ((m_s == m_t) & (a_s < a_t))
    best_ref[...] = jnp.where(take, a_s, a_t)


_merge_call = pl.pallas_call(
    _merge_body,
    out_shape=jax.ShapeDtypeStruct((1, N), jnp.int32),
)


BPW = N // NW


@functools.cache
def _make_gather_sc():
    mesh = plsc.VectorSubcoreMesh(core_axis_name="c", subcore_axis_name="s")

    @functools.partial(
        pl.kernel,
        mesh=mesh,
        out_type=jax.ShapeDtypeStruct((N, DIM), jnp.float32),
        scratch_types=[
            pltpu.VMEM((BPW,), jnp.int32),
            pltpu.VMEM((BPW, DIM), jnp.float32),
            pltpu.SemaphoreType.DMA,
        ],
        compiler_params=pltpu.CompilerParams(use_tc_tiling_on_sc=False),
    )
    def _gather_sc(table_hbm, idx_hbm, out_hbm, idx_v, rows_v, sem):
        wid = lax.axis_index("s") * NC + lax.axis_index("c")
        base = wid * BPW
        pltpu.sync_copy(idx_hbm.at[pl.ds(base, BPW)], idx_v)
        pltpu.async_copy(table_hbm.at[idx_v], rows_v, sem).wait()
        pltpu.sync_copy(rows_v, out_hbm.at[pl.ds(base, BPW)])

    return _gather_sc


def kernel(scores, emb_weight):
    st = jnp.swapaxes(scores, 0, 1)
    a_t, m_t = _tct_scan(st)
    m_flat, a_flat = _make_scan_sc()(st)
    mp = m_flat.reshape(NW, N)
    ap = a_flat.reshape(NW, N)
    best = _merge_call(a_t, m_t, mp, ap).reshape(N)
    emb = _make_gather_sc()(emb_weight, best)
    return emb, best
